# baseline (speedup 1.0000x reference)
import jax
import jax.numpy as jnp
from jax import lax
from jax.experimental import pallas as pl
from jax.experimental.pallas import tpu as pltpu
from jax.experimental.pallas import tpu_sc as plsc

N_NODES = 10000
N_PAD = 10240
D_IN = 128
D_OUT = 64
N_EDGES = 320000
NW = 32
CHUNK = 80
CPW = N_EDGES // (NW * CHUNK)
ROWS_PER_TILE = N_PAD // 16


def _mm2_body(x_ref, wl_ref, wr_ref, xl_ref, xr_ref):
    x = x_ref[...]
    h = x.shape[0] // 2
    xl = jnp.dot(x, wl_ref[...], preferred_element_type=jnp.float32)
    xr = jnp.dot(x, wr_ref[...], preferred_element_type=jnp.float32)
    xl_ref[:, 0:D_OUT] = xl[0:h]
    xl_ref[:, D_OUT:2 * D_OUT] = xl[h:2 * h]
    xr_ref[:, 0:D_OUT] = xr[0:h]
    xr_ref[:, D_OUT:2 * D_OUT] = xr[h:2 * h]


def _matmuls(node, Wl, Wr):
    blk = 2000
    return pl.pallas_call(
        _mm2_body,
        grid=(N_NODES // blk,),
        in_specs=[
            pl.BlockSpec((blk, D_IN), lambda i: (i, 0)),
            pl.BlockSpec((D_IN, D_OUT), lambda i: (0, 0)),
            pl.BlockSpec((D_IN, D_OUT), lambda i: (0, 0)),
        ],
        out_specs=[
            pl.BlockSpec((blk // 2, 2 * D_OUT), lambda i: (i, 0)),
            pl.BlockSpec((blk // 2, 2 * D_OUT), lambda i: (i, 0)),
        ],
        out_shape=[
            jax.ShapeDtypeStruct((N_NODES // 2, 2 * D_OUT), jnp.float32),
            jax.ShapeDtypeStruct((N_NODES // 2, 2 * D_OUT), jnp.float32),
        ],
    )(node, Wl, Wr)


def _edge_kernel_body(xl_hbm, xr_hbm, ei_hbm, att_hbm,
                      acc_hbm, den_hbm,
                      srcv, dstv, dsts, xlv, xrv, msgv, pwv, attv,
                      si0, si1, si2, sg0, sg1, sg2, ss0, ss1, ss2,
                      acc_sh, den_sh):
    semi = (si0, si1, si2)
    semg = (sg0, sg1, sg2)
    sems = (ss0, ss1, ss2)
    cid = lax.axis_index("c")
    sid = lax.axis_index("s")
    wid = cid * 16 + sid

    pltpu.sync_copy(att_hbm, attv)

    zf = jnp.zeros((16,), jnp.float32)

    def _zrow(r, _):
        for c4 in range(D_OUT // 16):
            msgv[0][r, pl.ds(c4 * 16, 16)] = zf
        pwv[0][r, pl.ds(0, 16)] = zf
        return 0

    lax.fori_loop(0, CHUNK, _zrow, 0)

    for b in range(ROWS_PER_TILE // CHUNK):
        pltpu.sync_copy(msgv[0], acc_sh.at[pl.ds(sid * ROWS_PER_TILE + b * CHUNK, CHUNK)])
    for b in range(ROWS_PER_TILE // CHUNK):
        pltpu.sync_copy(pwv[0], den_sh.at[pl.ds(sid * ROWS_PER_TILE + b * CHUNK, CHUNK)])

    plsc.subcore_barrier()

    att_c = [attv[pl.ds(c4 * 16, 16)] for c4 in range(D_OUT // 16)]
    iota16 = lax.iota(jnp.int32, 16)
    e0 = wid * CPW * CHUNK

    def issue_idx(c, b):
        base = jnp.minimum(e0 + c * CHUNK, N_EDGES - CHUNK)
        pltpu.async_copy(ei_hbm.at[0, pl.ds(base, CHUNK)], srcv[b], semi[b])
        pltpu.async_copy(ei_hbm.at[1, pl.ds(base, CHUNK)], dstv[b], semi[b])

    def issue_gathers(b):
        pltpu.make_async_copy(ei_hbm.at[0, pl.ds(0, CHUNK)], srcv[b], semi[b]).wait()
        pltpu.make_async_copy(ei_hbm.at[1, pl.ds(0, CHUNK)], dstv[b], semi[b]).wait()
        pltpu.async_copy(xl_hbm.at[srcv[b]], xlv[b], semg[b])
        pltpu.async_copy(xr_hbm.at[dstv[b]], xrv[b], semg[b])

    def wait_gathers(b):
        pltpu.make_async_copy(xl_hbm.at[srcv[b]], xlv[b], semg[b]).wait()
        pltpu.make_async_copy(xr_hbm.at[dstv[b]], xrv[b], semg[b]).wait()

    def save_dst(b):
        for k in range(CHUNK // 16):
            dsts[b][pl.ds(k * 16, 16)] = dstv[b][pl.ds(k * 16, 16)]

    def issue_scatter(b):
        pltpu.async_copy(msgv[b], acc_sh.at[dsts[b]], sems[b], add=True)
        pltpu.async_copy(pwv[b], den_sh.at[dsts[b]], sems[b], add=True)

    def wait_scatter(b):
        pltpu.make_async_copy(msgv[b], acc_sh.at[dsts[b]], sems[b]).wait()
        pltpu.make_async_copy(pwv[b], den_sh.at[dsts[b]], sems[b]).wait()

    def compute(b):
        def _group(g, _g):
            row0 = g * 16
            alpha = jnp.zeros((16,), jnp.float32)
            for j in range(16):
                r = row0 + j
                acc = None
                for c4 in range(D_OUT // 16):
                    e = xlv[b][r, pl.ds(c4 * 16, 16)] + xrv[b][r, pl.ds(c4 * 16, 16)]
                    e = jnp.maximum(e, 0.2 * e)
                    t = att_c[c4] * e
                    acc = t if acc is None else acc + t
                alpha = jnp.where(iota16 == j, jnp.sum(acc), alpha)
            p16 = jnp.exp(alpha)
            for j in range(16):
                r = row0 + j
                pj = p16[j]
                pj16 = jnp.broadcast_to(pj, (16,))
                pwv[b][r, pl.ds(0, 16)] = pj16
                for c4 in range(D_OUT // 16):
                    msgv[b][r, pl.ds(c4 * 16, 16)] = pj * xlv[b][r, pl.ds(c4 * 16, 16)]
            return 0

        lax.fori_loop(0, CHUNK // 16, _group, 0)

    def steady(c, b, first):
        if not first:
            wait_scatter((b + 1) % 3)
        wait_gathers(b)
        save_dst(b)
        issue_idx(c + 3, b)
        issue_gathers((b + 2) % 3)
        compute(b)
        issue_scatter(b)

    issue_idx(0, 0)
    issue_idx(1, 1)
    issue_idx(2, 2)
    issue_gathers(0)
    issue_gathers(1)
    steady(0, 0, True)
    steady(1, 1, True)

    def _main(t, _):
        c = 2 + t * 3
        steady(c + 0, 2, False)
        steady(c + 1, 0, False)
        steady(c + 2, 1, False)
        return 0

    lax.fori_loop(0, (CPW - 2) // 3, _main, 0)

    wait_gathers(CPW % 3)
    wait_gathers((CPW + 1) % 3)
    pltpu.make_async_copy(ei_hbm.at[0, pl.ds(0, CHUNK)],
                          srcv[(CPW + 2) % 3], semi[(CPW + 2) % 3]).wait()
    pltpu.make_async_copy(ei_hbm.at[1, pl.ds(0, CHUNK)],
                          dstv[(CPW + 2) % 3], semi[(CPW + 2) % 3]).wait()
    wait_scatter((CPW - 2) % 3)
    wait_scatter((CPW - 1) % 3)

    plsc.subcore_barrier()

    sl = pl.ds(sid * ROWS_PER_TILE, ROWS_PER_TILE)
    pltpu.sync_copy(acc_sh.at[sl], acc_hbm.at[cid].at[sl])
    pltpu.sync_copy(den_sh.at[sl], den_hbm.at[cid].at[sl])


def _edge_pass(xl, xr, edge_index, att):
    mesh = plsc.VectorSubcoreMesh(core_axis_name="c", subcore_axis_name="s")
    kern = pl.kernel(
        _edge_kernel_body,
        mesh=mesh,
        compiler_params=pltpu.CompilerParams(
            needs_layout_passes=False, use_tc_tiling_on_sc=False),
        out_type=[
            jax.ShapeDtypeStruct((2, N_PAD, D_OUT), jnp.float32),
            jax.ShapeDtypeStruct((2, N_PAD, 16), jnp.float32),
        ],
        scratch_types=[
            [pltpu.VMEM((CHUNK,), jnp.int32)] * 3,
            [pltpu.VMEM((CHUNK,), jnp.int32)] * 3,
            [pltpu.VMEM((CHUNK,), jnp.int32)] * 3,
            [pltpu.VMEM((CHUNK, D_OUT), jnp.float32)] * 3,
            [pltpu.VMEM((CHUNK, D_OUT), jnp.float32)] * 3,
            [pltpu.VMEM((CHUNK, D_OUT), jnp.float32)] * 3,
            [pltpu.VMEM((CHUNK, 16), jnp.float32)] * 3,
            pltpu.VMEM((D_OUT,), jnp.float32),
            pltpu.SemaphoreType.DMA, pltpu.SemaphoreType.DMA,
            pltpu.SemaphoreType.DMA, pltpu.SemaphoreType.DMA,
            pltpu.SemaphoreType.DMA, pltpu.SemaphoreType.DMA,
            pltpu.SemaphoreType.DMA, pltpu.SemaphoreType.DMA,
            pltpu.SemaphoreType.DMA,
            pltpu.VMEM_SHARED((N_PAD, D_OUT), jnp.float32),
            pltpu.VMEM_SHARED((N_PAD, 16), jnp.float32),
        ],
    )
    return kern(xl, xr, edge_index, att)


@jax.jit
def kernel(node, edge_index, Wl, Wr, att, bias, Wlin, blin):
    W2 = 2 * D_OUT
    xl2, xr2 = _matmuls(node, Wl, Wr)
    q = edge_index // 1000
    rem = edge_index - q * 1000
    ei_r = (q // 2) * 2000 + 2 * rem + (q % 2)
    acc, den = _edge_pass(xl2.reshape(N_NODES, D_OUT),
                          xr2.reshape(N_NODES, D_OUT), ei_r, att)
    acc2 = acc.reshape(2, N_PAD // 2, W2)
    den2 = den.reshape(2, N_PAD // 8, W2)

    bsum = jnp.tile((bias + blin).reshape(1, D_OUT), (1, 2))
    att2 = jnp.tile(att.reshape(1, D_OUT), (1, 2))

    def fin_body(acc0_ref, acc1_ref, den0_ref, den1_ref, x_ref, xl_ref,
                 xr_ref, wlin_ref, att_ref, b_ref, out_ref):
        xlb = xl_ref[...]
        half = xlb.shape[0]
        e = xlb + xr_ref[...]
        e = jnp.maximum(e, 0.2 * e)
        t = (e * att_ref[...]).reshape(half, 2, D_OUT)
        p3 = jnp.exp(jnp.sum(t, axis=2, keepdims=True))
        i = pl.program_id(0)
        w = pl.ds(i * (half // 4), half // 4)
        d128 = den0_ref[0, w, :] + den1_ref[0, w, :]
        dn = d128.reshape(half // 4, 8, 16)[:, :, :1].reshape(half, 2, 1)
        den_ = jnp.maximum(dn + p3, 1e-16)
        acc3 = (acc0_ref[0] + acc1_ref[0]).reshape(half, 2, D_OUT)
        xl3 = xlb.reshape(half, 2, D_OUT)
        gat3 = (acc3 + p3 * xl3) / den_
        lin = jnp.dot(x_ref[...], wlin_ref[...],
                      preferred_element_type=jnp.float32)
        lin3 = jnp.concatenate(
            [lin[0:half][:, None, :], lin[half:2 * half][:, None, :]], axis=1)
        out3 = gat3 + lin3 + b_ref[...].reshape(1, 2, D_OUT)
        out3 = jnp.maximum(out3, 0.0)
        out_ref[...] = jnp.concatenate([out3[:, 0, :], out3[:, 1, :]], axis=0)

    blk = 2000
    outp = pl.pallas_call(
        fin_body,
        grid=(N_NODES // blk,),
        in_specs=[
            pl.BlockSpec((1, blk // 2, W2), lambda i: (0, i, 0)),
            pl.BlockSpec((1, blk // 2, W2), lambda i: (1, i, 0)),
            pl.BlockSpec((1, N_PAD // 8, W2), lambda i: (0, 0, 0)),
            pl.BlockSpec((1, N_PAD // 8, W2), lambda i: (1, 0, 0)),
            pl.BlockSpec((blk, D_IN), lambda i: (i, 0)),
            pl.BlockSpec((blk // 2, W2), lambda i: (i, 0)),
            pl.BlockSpec((blk // 2, W2), lambda i: (i, 0)),
            pl.BlockSpec((D_IN, D_OUT), lambda i: (0, 0)),
            pl.BlockSpec((1, W2), lambda i: (0, 0)),
            pl.BlockSpec((1, W2), lambda i: (0, 0)),
        ],
        out_specs=pl.BlockSpec((blk, D_OUT), lambda i: (i, 0)),
        out_shape=jax.ShapeDtypeStruct((N_NODES, D_OUT), jnp.float32),
    )(acc2, acc2, den2, den2, node, xl2, xr2, Wlin, att2, bsum)
    return outp


# speedup vs baseline: 1.3550x; 1.3550x over previous
import jax
import jax.numpy as jnp
from jax import lax
from jax.experimental import pallas as pl
from jax.experimental.pallas import tpu as pltpu
from jax.experimental.pallas import tpu_sc as plsc

N_NODES = 10000
N_PAD = 10240
D_IN = 128
D_OUT = 64
N_EDGES = 320000
NW = 32
CHUNK = 80
CPW = N_EDGES // (NW * CHUNK)
ROWS_PER_TILE = N_PAD // 16


def _mm2_body(x_ref, wl_ref, wr_ref, xl_ref, xr_ref):
    x = x_ref[...]
    h = x.shape[0] // 2
    xl = jnp.dot(x, wl_ref[...], preferred_element_type=jnp.float32)
    xr = jnp.dot(x, wr_ref[...], preferred_element_type=jnp.float32)
    xl_ref[:, 0:D_OUT] = xl[0:h]
    xl_ref[:, D_OUT:2 * D_OUT] = xl[h:2 * h]
    xr_ref[:, 0:D_OUT] = xr[0:h]
    xr_ref[:, D_OUT:2 * D_OUT] = xr[h:2 * h]


def _matmuls(node, Wl, Wr):
    blk = 2000
    return pl.pallas_call(
        _mm2_body,
        grid=(N_NODES // blk,),
        in_specs=[
            pl.BlockSpec((blk, D_IN), lambda i: (i, 0)),
            pl.BlockSpec((D_IN, D_OUT), lambda i: (0, 0)),
            pl.BlockSpec((D_IN, D_OUT), lambda i: (0, 0)),
        ],
        out_specs=[
            pl.BlockSpec((blk // 2, 2 * D_OUT), lambda i: (i, 0)),
            pl.BlockSpec((blk // 2, 2 * D_OUT), lambda i: (i, 0)),
        ],
        out_shape=[
            jax.ShapeDtypeStruct((N_NODES // 2, 2 * D_OUT), jnp.float32),
            jax.ShapeDtypeStruct((N_NODES // 2, 2 * D_OUT), jnp.float32),
        ],
    )(node, Wl, Wr)


def _edge_kernel_body(xl_hbm, xr_hbm, ei_hbm, att_hbm,
                      acc_hbm, den_hbm,
                      srcv, dstv, dsts, xlv, xrv, msgv, pwv, dexp, attv,
                      si0, si1, si2, sg0, sg1, sg2, ss0, ss1, ss2,
                      acc_sh, den_sh):
    semi = (si0, si1, si2)
    semg = (sg0, sg1, sg2)
    sems = (ss0, ss1, ss2)
    cid = lax.axis_index("c")
    sid = lax.axis_index("s")
    wid = cid * 16 + sid

    pltpu.sync_copy(att_hbm, attv)

    zf = jnp.zeros((16,), jnp.float32)

    def _zrow(r, _):
        for c4 in range(D_OUT // 16):
            msgv[0][r, pl.ds(c4 * 16, 16)] = zf
        return 0

    lax.fori_loop(0, CHUNK, _zrow, 0)
    for c8 in range(CHUNK // 16):
        pwv[0][pl.ds(c8 * 16, 16)] = zf

    for b in range(ROWS_PER_TILE // CHUNK):
        pltpu.sync_copy(msgv[0], acc_sh.at[pl.ds(sid * ROWS_PER_TILE + b * CHUNK, CHUNK)])
    for b in range(ROWS_PER_TILE // CHUNK):
        pltpu.sync_copy(pwv[0], den_sh.at[pl.ds(sid * ROWS_PER_TILE + b * CHUNK, CHUNK)])

    plsc.subcore_barrier()

    att_c = [attv[pl.ds(c4 * 16, 16)] for c4 in range(D_OUT // 16)]
    iota16 = lax.iota(jnp.int32, 16)
    e0 = wid * CPW * CHUNK

    def issue_idx(c, b):
        base = jnp.minimum(e0 + c * CHUNK, N_EDGES - CHUNK)
        pltpu.async_copy(ei_hbm.at[0, pl.ds(base, CHUNK)], srcv[b], semi[b])
        pltpu.async_copy(ei_hbm.at[1, pl.ds(base, CHUNK)], dstv[b], semi[b])

    def issue_gathers(b):
        pltpu.make_async_copy(ei_hbm.at[0, pl.ds(0, CHUNK)], srcv[b], semi[b]).wait()
        pltpu.make_async_copy(ei_hbm.at[1, pl.ds(0, CHUNK)], dstv[b], semi[b]).wait()
        pltpu.async_copy(xl_hbm.at[srcv[b]], xlv[b], semg[b])
        pltpu.async_copy(xr_hbm.at[dstv[b]], xrv[b], semg[b])

    def wait_gathers(b):
        pltpu.make_async_copy(xl_hbm.at[srcv[b]], xlv[b], semg[b]).wait()
        pltpu.make_async_copy(xr_hbm.at[dstv[b]], xrv[b], semg[b]).wait()

    def save_dst(b):
        for k in range(CHUNK // 16):
            dsts[b][pl.ds(k * 16, 16)] = dstv[b][pl.ds(k * 16, 16)]

    def issue_scatter(b):
        pltpu.async_copy(msgv[b], acc_sh.at[dsts[b]], sems[b], add=True)
        pltpu.async_copy(pwv[b], den_sh.at[dsts[b]], sems[b], add=True)

    def wait_scatter(b):
        pltpu.make_async_copy(msgv[b], acc_sh.at[dsts[b]], sems[b]).wait()
        pltpu.make_async_copy(pwv[b], den_sh.at[dsts[b]], sems[b]).wait()

    def compute(b):
        def _group(g, _g):
            row0 = g * 16
            alpha = jnp.zeros((16,), jnp.float32)
            for j in range(16):
                r = row0 + j
                acc = None
                for c4 in range(D_OUT // 16):
                    e = xlv[b][r, pl.ds(c4 * 16, 16)] + xrv[b][r, pl.ds(c4 * 16, 16)]
                    e = jnp.maximum(e, 0.2 * e)
                    t = att_c[c4] * e
                    acc = t if acc is None else acc + t
                alpha = jnp.where(iota16 == j, jnp.sum(acc), alpha)
            p16 = jnp.exp(alpha)
            for j in range(16):
                r = row0 + j
                pj = p16[j]
            pwv[b][pl.ds(row0, 16)] = p16
            for j in range(16):
                r = row0 + j
                pj = p16[j]
                for c4 in range(D_OUT // 16):
                    msgv[b][r, pl.ds(c4 * 16, 16)] = pj * xlv[b][r, pl.ds(c4 * 16, 16)]
            return 0

        lax.fori_loop(0, CHUNK // 16, _group, 0)

    def steady(c, b, first):
        if not first:
            wait_scatter((b + 1) % 3)
        wait_gathers(b)
        save_dst(b)
        issue_idx(c + 3, b)
        issue_gathers((b + 2) % 3)
        compute(b)
        issue_scatter(b)

    issue_idx(0, 0)
    issue_idx(1, 1)
    issue_idx(2, 2)
    issue_gathers(0)
    issue_gathers(1)
    steady(0, 0, True)
    steady(1, 1, True)

    def _main(t, _):
        c = 2 + t * 3
        steady(c + 0, 2, False)
        steady(c + 1, 0, False)
        steady(c + 2, 1, False)
        return 0

    lax.fori_loop(0, (CPW - 2) // 3, _main, 0)

    wait_gathers(CPW % 3)
    wait_gathers((CPW + 1) % 3)
    pltpu.make_async_copy(ei_hbm.at[0, pl.ds(0, CHUNK)],
                          srcv[(CPW + 2) % 3], semi[(CPW + 2) % 3]).wait()
    pltpu.make_async_copy(ei_hbm.at[1, pl.ds(0, CHUNK)],
                          dstv[(CPW + 2) % 3], semi[(CPW + 2) % 3]).wait()
    wait_scatter((CPW - 2) % 3)
    wait_scatter((CPW - 1) % 3)

    plsc.subcore_barrier()

    sl = pl.ds(sid * ROWS_PER_TILE, ROWS_PER_TILE)
    pltpu.sync_copy(acc_sh.at[sl], acc_hbm.at[cid].at[sl])

    def _dgrp(g, _):
        for part in range(2):
            d16 = pwv[part][pl.ds(g * 16, 16)]
            for j in range(16):
                dexp[g * 16 + j + part * CHUNK, pl.ds(0, 16)] = (
                    jnp.broadcast_to(d16[j], (16,)))
        return 0

    for blk8 in range(ROWS_PER_TILE // (2 * CHUNK)):
        base = sid * ROWS_PER_TILE + blk8 * 2 * CHUNK
        pltpu.sync_copy(den_sh.at[pl.ds(base, CHUNK)], pwv[0])
        pltpu.sync_copy(den_sh.at[pl.ds(base + CHUNK, CHUNK)], pwv[1])
        lax.fori_loop(0, CHUNK // 16, _dgrp, 0)
        pltpu.sync_copy(dexp, den_hbm.at[cid].at[pl.ds(base, 2 * CHUNK)])


def _edge_pass(xl, xr, edge_index, att):
    mesh = plsc.VectorSubcoreMesh(core_axis_name="c", subcore_axis_name="s")
    kern = pl.kernel(
        _edge_kernel_body,
        mesh=mesh,
        compiler_params=pltpu.CompilerParams(
            needs_layout_passes=False, use_tc_tiling_on_sc=False),
        out_type=[
            jax.ShapeDtypeStruct((2, N_PAD, D_OUT), jnp.float32),
            jax.ShapeDtypeStruct((2, N_PAD, 16), jnp.float32),
        ],
        scratch_types=[
            [pltpu.VMEM((CHUNK,), jnp.int32)] * 3,
            [pltpu.VMEM((CHUNK,), jnp.int32)] * 3,
            [pltpu.VMEM((CHUNK,), jnp.int32)] * 3,
            [pltpu.VMEM((CHUNK, D_OUT), jnp.float32)] * 3,
            [pltpu.VMEM((CHUNK, D_OUT), jnp.float32)] * 3,
            [pltpu.VMEM((CHUNK, D_OUT), jnp.float32)] * 3,
            [pltpu.VMEM((CHUNK,), jnp.float32)] * 3,
            pltpu.VMEM((2 * CHUNK, 16), jnp.float32),
            pltpu.VMEM((D_OUT,), jnp.float32),
            pltpu.SemaphoreType.DMA, pltpu.SemaphoreType.DMA,
            pltpu.SemaphoreType.DMA, pltpu.SemaphoreType.DMA,
            pltpu.SemaphoreType.DMA, pltpu.SemaphoreType.DMA,
            pltpu.SemaphoreType.DMA, pltpu.SemaphoreType.DMA,
            pltpu.SemaphoreType.DMA,
            pltpu.VMEM_SHARED((N_PAD, D_OUT), jnp.float32),
            pltpu.VMEM_SHARED((N_PAD,), jnp.float32),
        ],
    )
    return kern(xl, xr, edge_index, att)


@jax.jit
def kernel(node, edge_index, Wl, Wr, att, bias, Wlin, blin):
    W2 = 2 * D_OUT
    xl2, xr2 = _matmuls(node, Wl, Wr)
    q = (edge_index.astype(jnp.float32) * (1.0 / 1000.0)).astype(jnp.int32)
    rem = edge_index - q * 1000
    ei_r = (q // 2) * 2000 + 2 * rem + (q % 2)
    acc, den = _edge_pass(xl2.reshape(N_NODES, D_OUT),
                          xr2.reshape(N_NODES, D_OUT), ei_r, att)
    acc2 = acc.reshape(2, N_PAD // 2, W2)
    den2 = den.reshape(2, N_PAD // 2, 32)

    bsum = jnp.tile((bias + blin).reshape(1, D_OUT), (1, 2))
    att2 = jnp.tile(att.reshape(1, D_OUT), (1, 2))
    i128 = jnp.arange(W2)
    Bm = ((i128[:, None] // D_OUT) == (i128[None, :] // D_OUT)
          ).astype(jnp.float32)
    i32_ = jnp.arange(32)
    Cm = (((i32_[:, None] // 16) == (i128[None, :] // D_OUT))
          & ((i32_[:, None] % 16) == (i128[None, :] % 16))
          ).astype(jnp.float32)

    def fin_body(acc0_ref, acc1_ref, den0_ref, den1_ref, x_ref, xl_ref,
                 xr_ref, wlin_ref, att_ref, b_ref, bm_ref, cm_ref, out_ref):
        xlb = xl_ref[...]
        half = xlb.shape[0]
        e = xlb + xr_ref[...]
        e = jnp.maximum(e, 0.2 * e)
        alpha = jnp.dot(e * att_ref[...], bm_ref[...],
                        preferred_element_type=jnp.float32)
        p2 = jnp.exp(alpha)
        dn = jnp.dot(den0_ref[0] + den1_ref[0], cm_ref[...],
                     preferred_element_type=jnp.float32)
        den_ = jnp.maximum(dn + p2, 1e-16)
        gat = (acc0_ref[0] + acc1_ref[0] + p2 * xlb) / den_
        lin = jnp.dot(x_ref[...], wlin_ref[...],
                      preferred_element_type=jnp.float32)
        lin2 = jnp.concatenate([lin[0:half], lin[half:2 * half]], axis=1)
        out2 = jnp.maximum(gat + lin2 + b_ref[...], 0.0)
        out_ref[...] = jnp.concatenate(
            [out2[:, 0:D_OUT], out2[:, D_OUT:W2]], axis=0)

    blk = 2000
    outp = pl.pallas_call(
        fin_body,
        grid=(N_NODES // blk,),
        in_specs=[
            pl.BlockSpec((1, blk // 2, W2), lambda i: (0, i, 0)),
            pl.BlockSpec((1, blk // 2, W2), lambda i: (1, i, 0)),
            pl.BlockSpec((1, blk // 2, 32), lambda i: (0, i, 0)),
            pl.BlockSpec((1, blk // 2, 32), lambda i: (1, i, 0)),
            pl.BlockSpec((blk, D_IN), lambda i: (i, 0)),
            pl.BlockSpec((blk // 2, W2), lambda i: (i, 0)),
            pl.BlockSpec((blk // 2, W2), lambda i: (i, 0)),
            pl.BlockSpec((D_IN, D_OUT), lambda i: (0, 0)),
            pl.BlockSpec((1, W2), lambda i: (0, 0)),
            pl.BlockSpec((1, W2), lambda i: (0, 0)),
            pl.BlockSpec((W2, W2), lambda i: (0, 0)),
            pl.BlockSpec((32, W2), lambda i: (0, 0)),
        ],
        out_specs=pl.BlockSpec((blk, D_OUT), lambda i: (i, 0)),
        out_shape=jax.ShapeDtypeStruct((N_NODES, D_OUT), jnp.float32),
    )(acc2, acc2, den2, den2, node, xl2, xr2, Wlin, att2, bsum, Bm, Cm)
    return outp
